# baseline (speedup 1.0000x reference)
import jax
import jax.numpy as jnp
from jax import lax
from jax.experimental import pallas as pl
from jax.experimental.pallas import tpu as pltpu
from jax.experimental.pallas import tpu_sc as plsc

N = 10000
E = 320000
Q = 65536
D = 128
NC = 2
NS = 16
NW = NC * NS
NPAD = 10240
RT = NPAD // NS
CH = 128
KE = 80
KQ = (Q // NW) // CH
QW = Q // NW
KQH = KQ // 2
QWH = QW // 2
QH = Q // 2


def _mm_body(emb_ref, x_ref, w_ref, o_ref):
    o_ref[...] = (
        jnp.dot(emb_ref[...], w_ref[0], preferred_element_type=jnp.float32)
        + jnp.dot(x_ref[...], w_ref[1], preferred_element_type=jnp.float32))


def _node_matmul(emb, x, gnn_w):
    blk = 400
    return pl.pallas_call(
        _mm_body,
        grid=(N // blk,),
        in_specs=[
            pl.BlockSpec((blk, D), lambda i: (i, 0)),
            pl.BlockSpec((blk, D), lambda i: (i, 0)),
            pl.BlockSpec((2, D, D), lambda i: (0, 0, 0)),
        ],
        out_specs=pl.BlockSpec((blk, D), lambda i: (i, 0)),
        out_shape=jax.ShapeDtypeStruct((NPAD, D), jnp.float32),
    )(emb, x, gnn_w.reshape(2, D, D))


def _seg_body(y_hbm, srcs, dsts, sum_out, deg_out,
              acc, degs, src_v, dst_v, msg0, msg1, zbuf, onesv, zv,
              sem0, sem1):
    c = lax.axis_index("c")
    s = lax.axis_index("s")
    wid = s * NC + c
    t = s

    zero16 = jnp.zeros((16,), jnp.float32)
    for i in range(8):
        for j in range(8):
            zbuf[i, pl.ds(j * 16, 16)] = zero16
    for i in range(RT // 16):
        zv[pl.ds(i * 16, 16)] = zero16
    for i in range(8):
        onesv[pl.ds(i * 16, 16)] = jnp.ones((16,), jnp.float32)
    for i in range(RT // 8):
        pltpu.sync_copy(zbuf, acc.at[pl.ds(t * RT + i * 8, 8)])
    pltpu.sync_copy(zv, degs.at[pl.ds(t * RT, RT)])
    plsc.subcore_barrier()

    KEH = KE // 2
    for p in range(2):
        pltpu.sync_copy(srcs.at[wid, pl.ds(p * KEH, KEH)], src_v)
        pltpu.sync_copy(dsts.at[wid, pl.ds(p * KEH, KEH)], dst_v)

        pltpu.async_copy(y_hbm.at[src_v.at[0]], msg0, sem0)
        pltpu.async_copy(y_hbm.at[src_v.at[1]], msg1, sem1)

        @pl.loop(0, KEH // 2)
        def _chunks(i):
            for b, (msg, sem) in enumerate(((msg0, sem0), (msg1, sem1))):
                j = 2 * i + b
                pltpu.make_async_copy(y_hbm.at[src_v.at[j]], msg, sem).wait()
                pltpu.sync_copy(msg, acc.at[dst_v.at[j]], add=True)
                pltpu.sync_copy(onesv, degs.at[dst_v.at[j]], add=True)

                @pl.when(j + 2 < KEH)
                def _():
                    pltpu.async_copy(y_hbm.at[src_v.at[j + 2]], msg, sem)

    plsc.subcore_barrier()
    pltpu.sync_copy(acc.at[pl.ds(t * RT, RT)],
                    sum_out.at[c, pl.ds(t * RT, RT)])
    pltpu.sync_copy(degs.at[pl.ds(t * RT, RT)],
                    deg_out.at[c, pl.ds(t * RT, RT)])


def _segment_sum(y, srcs, dsts):
    mesh = plsc.VectorSubcoreMesh(core_axis_name="c", subcore_axis_name="s",
                                  num_cores=NC, num_subcores=NS)
    f = pl.kernel(
        _seg_body,
        out_type=[
            jax.ShapeDtypeStruct((NC, NPAD, D), jnp.float32),
            jax.ShapeDtypeStruct((NC, NPAD), jnp.float32),
        ],
        mesh=mesh,
        scratch_types=[
            pltpu.VMEM_SHARED((NPAD, D), jnp.float32),
            pltpu.VMEM_SHARED((NPAD,), jnp.float32),
            pltpu.VMEM((KE // 2, CH), jnp.int32),
            pltpu.VMEM((KE // 2, CH), jnp.int32),
            pltpu.VMEM((CH, D), jnp.float32),
            pltpu.VMEM((CH, D), jnp.float32),
            pltpu.VMEM((8, D), jnp.float32),
            pltpu.VMEM((CH,), jnp.float32),
            pltpu.VMEM((RT,), jnp.float32),
            pltpu.SemaphoreType.DMA,
            pltpu.SemaphoreType.DMA,
        ],
    )
    return f(y, srcs, dsts)


def _merge_body(s0_ref, s1_ref, d0_ref, d1_ref, b_ref, h_ref):
    deg = jnp.maximum(d0_ref[...] + d1_ref[...], 1.0)
    h_ref[...] = jnp.maximum(
        (s0_ref[...] + s1_ref[...]) / deg + b_ref[...], 0.0)


def _merge(sums, degs, gnn_b):
    blk = 512
    d2 = degs.reshape(NC, NPAD, 1)
    return pl.pallas_call(
        _merge_body,
        grid=(NPAD // blk,),
        in_specs=[
            pl.BlockSpec((blk, D), lambda i: (i, 0)),
            pl.BlockSpec((blk, D), lambda i: (i, 0)),
            pl.BlockSpec((blk, 1), lambda i: (i, 0)),
            pl.BlockSpec((blk, 1), lambda i: (i, 0)),
            pl.BlockSpec((1, D), lambda i: (0, 0)),
        ],
        out_specs=pl.BlockSpec((blk, D), lambda i: (i, 0)),
        out_shape=jax.ShapeDtypeStruct((NPAD, D), jnp.float32),
    )(sums[0], sums[1], d2[0], d2[1], gnn_b.reshape(1, D))


def _gatherq_body(h_hbm, e0s, e1s, hs_out, hd_out,
                  e0_v, e1_v, hs0, hs1, hd0, hd1,
                  sa0, sa1, sb0, sb1):
    c = lax.axis_index("c")
    s = lax.axis_index("s")
    wid = s * NC + c

    pltpu.sync_copy(e0s.at[wid], e0_v)
    pltpu.sync_copy(e1s.at[wid], e1_v)

    pltpu.async_copy(h_hbm.at[e0_v.at[0]], hs0, sa0)
    pltpu.async_copy(h_hbm.at[e1_v.at[0]], hd0, sb0)
    pltpu.async_copy(h_hbm.at[e0_v.at[1]], hs1, sa1)
    pltpu.async_copy(h_hbm.at[e1_v.at[1]], hd1, sb1)

    @pl.loop(0, KQH // 2)
    def _chunks(i):
        for b, (hs, hd, sa, sb) in enumerate(
                ((hs0, hd0, sa0, sb0), (hs1, hd1, sa1, sb1))):
            j = 2 * i + b
            base = wid * QWH + j * CH
            pltpu.make_async_copy(h_hbm.at[e0_v.at[j]], hs, sa).wait()
            pltpu.make_async_copy(h_hbm.at[e1_v.at[j]], hd, sb).wait()
            pltpu.sync_copy(hs, hs_out.at[pl.ds(base, CH)])
            pltpu.sync_copy(hd, hd_out.at[pl.ds(base, CH)])

            @pl.when(j + 2 < KQH)
            def _():
                pltpu.async_copy(h_hbm.at[e0_v.at[j + 2]], hs, sa)
                pltpu.async_copy(h_hbm.at[e1_v.at[j + 2]], hd, sb)


def _gatherq(h, e0s, e1s):
    mesh = plsc.VectorSubcoreMesh(core_axis_name="c", subcore_axis_name="s",
                                  num_cores=NC, num_subcores=NS)
    f = pl.kernel(
        _gatherq_body,
        out_type=[
            jax.ShapeDtypeStruct((QH, D), jnp.float32),
            jax.ShapeDtypeStruct((QH, D), jnp.float32),
        ],
        mesh=mesh,
        scratch_types=[
            pltpu.VMEM((KQH, CH), jnp.int32),
            pltpu.VMEM((KQH, CH), jnp.int32),
            pltpu.VMEM((CH, D), jnp.float32),
            pltpu.VMEM((CH, D), jnp.float32),
            pltpu.VMEM((CH, D), jnp.float32),
            pltpu.VMEM((CH, D), jnp.float32),
            pltpu.SemaphoreType.DMA,
            pltpu.SemaphoreType.DMA,
            pltpu.SemaphoreType.DMA,
            pltpu.SemaphoreType.DMA,
        ],
    )
    return f(h, e0s, e1s)


def _score_body(hs_ref, hd_ref, w_ref, b_ref, o_ref):
    z = jnp.sum(hs_ref[...] * hd_ref[...] * w_ref[...], axis=1,
                keepdims=True) + b_ref[...]
    o_ref[...] = 1.0 / (1.0 + jnp.exp(-z))


def _score(hs, hd, pred_w, pred_b):
    blk = 8192
    return pl.pallas_call(
        _score_body,
        grid=(QH // blk,),
        in_specs=[
            pl.BlockSpec((blk, D), lambda i: (i, 0)),
            pl.BlockSpec((blk, D), lambda i: (i, 0)),
            pl.BlockSpec((1, D), lambda i: (0, 0)),
            pl.BlockSpec((1, 1), lambda i: (0, 0)),
        ],
        out_specs=pl.BlockSpec((blk, 1), lambda i: (i, 0)),
        out_shape=jax.ShapeDtypeStruct((QH, 1), jnp.float32),
    )(hs, hd, pred_w.reshape(1, D), pred_b.reshape(1, 1))


@jax.jit
def kernel(x, edges, adj, emb_weight, gnn_w, gnn_b, pred_w, pred_b):
    y = _node_matmul(emb_weight, x.astype(jnp.float32), gnn_w)

    epad = NW * KE * CH - E
    srcs = jnp.concatenate(
        [adj[0].astype(jnp.int32), jnp.arange(epad, dtype=jnp.int32) % N]
    ).reshape(NW, KE, CH)
    dsts = jnp.concatenate(
        [adj[1].astype(jnp.int32), jnp.full((epad,), N, jnp.int32)]
    ).reshape(NW, KE, CH)

    sums, degs = _segment_sum(y, srcs, dsts)

    h = _merge(sums, degs, gnn_b)

    e0s = edges[0].astype(jnp.int32).reshape(NW, KQ, CH)
    e1s = edges[1].astype(jnp.int32).reshape(NW, KQ, CH)

    halves = []
    for p in range(2):
        sl = slice(p * KQH, (p + 1) * KQH)
        hs, hd = _gatherq(h, e0s[:, sl], e1s[:, sl])
        halves.append(_score(hs, hd, pred_w[:, 0], pred_b)[:, 0])

    s = jnp.stack(halves).reshape(2, NW, QWH)
    return s.transpose(1, 0, 2).reshape(Q)


# speedup vs baseline: 1.0042x; 1.0042x over previous
import jax
import jax.numpy as jnp
from jax import lax
from jax.experimental import pallas as pl
from jax.experimental.pallas import tpu as pltpu
from jax.experimental.pallas import tpu_sc as plsc

N = 10000
E = 320000
Q = 65536
D = 128
NC = 2
NS = 16
NW = NC * NS
NPAD = 10240
RT = NPAD // NS
CH = 128
KE = 80
KQ = (Q // NW) // CH
QW = Q // NW
KQH = KQ // 2
QWH = QW // 2
QH = Q // 2


def _mm_body(emb_ref, x_ref, w_ref, o_ref):
    o_ref[...] = (
        jnp.dot(emb_ref[...], w_ref[0], preferred_element_type=jnp.float32)
        + jnp.dot(x_ref[...], w_ref[1], preferred_element_type=jnp.float32))


def _node_matmul(emb, x, gnn_w):
    blk = 400
    return pl.pallas_call(
        _mm_body,
        grid=(N // blk,),
        in_specs=[
            pl.BlockSpec((blk, D), lambda i: (i, 0)),
            pl.BlockSpec((blk, D), lambda i: (i, 0)),
            pl.BlockSpec((2, D, D), lambda i: (0, 0, 0)),
        ],
        out_specs=pl.BlockSpec((blk, D), lambda i: (i, 0)),
        out_shape=jax.ShapeDtypeStruct((NPAD, D), jnp.float32),
    )(emb, x, gnn_w.reshape(2, D, D))


def _seg_body(y_hbm, srcs, dsts, sum_out, deg_out,
              acc, degs, src_v, dst_v, msg0, msg1, zbuf, onesv, zv,
              sem0, sem1):
    c = lax.axis_index("c")
    s = lax.axis_index("s")
    wid = s * NC + c
    t = s

    zero16 = jnp.zeros((16,), jnp.float32)
    for i in range(8):
        for j in range(8):
            zbuf[i, pl.ds(j * 16, 16)] = zero16
    for i in range(RT // 16):
        zv[pl.ds(i * 16, 16)] = zero16
    for i in range(8):
        onesv[pl.ds(i * 16, 16)] = jnp.ones((16,), jnp.float32)
    for i in range(RT // 8):
        pltpu.sync_copy(zbuf, acc.at[pl.ds(t * RT + i * 8, 8)])
    pltpu.sync_copy(zv, degs.at[pl.ds(t * RT, RT)])
    plsc.subcore_barrier()

    KEH = KE // 2
    for p in range(2):
        pltpu.sync_copy(srcs.at[wid, pl.ds(p * KEH, KEH)], src_v)
        pltpu.sync_copy(dsts.at[wid, pl.ds(p * KEH, KEH)], dst_v)

        pltpu.async_copy(y_hbm.at[src_v.at[0]], msg0, sem0)
        pltpu.async_copy(y_hbm.at[src_v.at[1]], msg1, sem1)

        @pl.loop(0, KEH // 2)
        def _chunks(i):
            for b, (msg, sem) in enumerate(((msg0, sem0), (msg1, sem1))):
                j = 2 * i + b
                pltpu.make_async_copy(y_hbm.at[src_v.at[j]], msg, sem).wait()
                pltpu.sync_copy(msg, acc.at[dst_v.at[j]], add=True)
                pltpu.sync_copy(onesv, degs.at[dst_v.at[j]], add=True)

                @pl.when(j + 2 < KEH)
                def _():
                    pltpu.async_copy(y_hbm.at[src_v.at[j + 2]], msg, sem)

    plsc.subcore_barrier()
    pltpu.sync_copy(acc.at[pl.ds(t * RT, RT)],
                    sum_out.at[c, pl.ds(t * RT, RT)])
    pltpu.sync_copy(degs.at[pl.ds(t * RT, RT)],
                    deg_out.at[c, pl.ds(t * RT, RT)])


def _segment_sum(y, srcs, dsts):
    mesh = plsc.VectorSubcoreMesh(core_axis_name="c", subcore_axis_name="s",
                                  num_cores=NC, num_subcores=NS)
    f = pl.kernel(
        _seg_body,
        out_type=[
            jax.ShapeDtypeStruct((NC, NPAD, D), jnp.float32),
            jax.ShapeDtypeStruct((NC, NPAD), jnp.float32),
        ],
        mesh=mesh,
        scratch_types=[
            pltpu.VMEM_SHARED((NPAD, D), jnp.float32),
            pltpu.VMEM_SHARED((NPAD,), jnp.float32),
            pltpu.VMEM((KE // 2, CH), jnp.int32),
            pltpu.VMEM((KE // 2, CH), jnp.int32),
            pltpu.VMEM((CH, D), jnp.float32),
            pltpu.VMEM((CH, D), jnp.float32),
            pltpu.VMEM((8, D), jnp.float32),
            pltpu.VMEM((CH,), jnp.float32),
            pltpu.VMEM((RT,), jnp.float32),
            pltpu.SemaphoreType.DMA,
            pltpu.SemaphoreType.DMA,
        ],
    )
    return f(y, srcs, dsts)


def _merge_body(s0_ref, s1_ref, d0_ref, d1_ref, b_ref, h_ref):
    deg = jnp.maximum(d0_ref[...] + d1_ref[...], 1.0)
    h_ref[...] = jnp.maximum(
        (s0_ref[...] + s1_ref[...]) / deg + b_ref[...], 0.0)


def _merge(sums, degs, gnn_b):
    blk = 512
    d2 = degs.reshape(NC, NPAD, 1)
    return pl.pallas_call(
        _merge_body,
        grid=(NPAD // blk,),
        in_specs=[
            pl.BlockSpec((blk, D), lambda i: (i, 0)),
            pl.BlockSpec((blk, D), lambda i: (i, 0)),
            pl.BlockSpec((blk, 1), lambda i: (i, 0)),
            pl.BlockSpec((blk, 1), lambda i: (i, 0)),
            pl.BlockSpec((1, D), lambda i: (0, 0)),
        ],
        out_specs=pl.BlockSpec((blk, D), lambda i: (i, 0)),
        out_shape=jax.ShapeDtypeStruct((NPAD, D), jnp.float32),
    )(sums[0], sums[1], d2[0], d2[1], gnn_b.reshape(1, D))


def _gatherq_body(h_hbm, e0s, e1s, hs_out, hd_out,
                  e0_v, e1_v, hs0, hs1, hd0, hd1,
                  sa0, sa1, sb0, sb1):
    c = lax.axis_index("c")
    s = lax.axis_index("s")
    wid = s * NC + c

    pltpu.sync_copy(e0s.at[wid], e0_v)
    pltpu.sync_copy(e1s.at[wid], e1_v)

    pltpu.async_copy(h_hbm.at[e0_v.at[0]], hs0, sa0)
    pltpu.async_copy(h_hbm.at[e1_v.at[0]], hd0, sb0)
    pltpu.async_copy(h_hbm.at[e0_v.at[1]], hs1, sa1)
    pltpu.async_copy(h_hbm.at[e1_v.at[1]], hd1, sb1)

    @pl.loop(0, KQ // 2)
    def _chunks(i):
        for b, (hs, hd, sa, sb) in enumerate(
                ((hs0, hd0, sa0, sb0), (hs1, hd1, sa1, sb1))):
            j = 2 * i + b
            base = wid * QW + j * CH
            pltpu.make_async_copy(h_hbm.at[e0_v.at[j]], hs, sa).wait()
            pltpu.make_async_copy(h_hbm.at[e1_v.at[j]], hd, sb).wait()
            pltpu.sync_copy(hs, hs_out.at[pl.ds(base, CH)])
            pltpu.sync_copy(hd, hd_out.at[pl.ds(base, CH)])

            @pl.when(j + 2 < KQ)
            def _():
                pltpu.async_copy(h_hbm.at[e0_v.at[j + 2]], hs, sa)
                pltpu.async_copy(h_hbm.at[e1_v.at[j + 2]], hd, sb)


def _gatherq(h, e0s, e1s):
    mesh = plsc.VectorSubcoreMesh(core_axis_name="c", subcore_axis_name="s",
                                  num_cores=NC, num_subcores=NS)
    f = pl.kernel(
        _gatherq_body,
        out_type=[
            jax.ShapeDtypeStruct((Q, D), jnp.float32),
            jax.ShapeDtypeStruct((Q, D), jnp.float32),
        ],
        mesh=mesh,
        scratch_types=[
            pltpu.VMEM((KQ, CH), jnp.int32),
            pltpu.VMEM((KQ, CH), jnp.int32),
            pltpu.VMEM((CH, D), jnp.float32),
            pltpu.VMEM((CH, D), jnp.float32),
            pltpu.VMEM((CH, D), jnp.float32),
            pltpu.VMEM((CH, D), jnp.float32),
            pltpu.SemaphoreType.DMA,
            pltpu.SemaphoreType.DMA,
            pltpu.SemaphoreType.DMA,
            pltpu.SemaphoreType.DMA,
        ],
    )
    return f(h, e0s, e1s)


def _score_body(hs_ref, hd_ref, w_ref, b_ref, o_ref):
    z = jnp.sum(hs_ref[...] * hd_ref[...] * w_ref[...], axis=1,
                keepdims=True) + b_ref[...]
    o_ref[...] = 1.0 / (1.0 + jnp.exp(-z))


def _score(hs, hd, pred_w, pred_b):
    blk = 16384
    return pl.pallas_call(
        _score_body,
        grid=(Q // blk,),
        in_specs=[
            pl.BlockSpec((blk, D), lambda i: (i, 0)),
            pl.BlockSpec((blk, D), lambda i: (i, 0)),
            pl.BlockSpec((1, D), lambda i: (0, 0)),
            pl.BlockSpec((1, 1), lambda i: (0, 0)),
        ],
        out_specs=pl.BlockSpec((blk, 1), lambda i: (i, 0)),
        out_shape=jax.ShapeDtypeStruct((Q, 1), jnp.float32),
    )(hs, hd, pred_w.reshape(1, D), pred_b.reshape(1, 1))


@jax.jit
def kernel(x, edges, adj, emb_weight, gnn_w, gnn_b, pred_w, pred_b):
    y = _node_matmul(emb_weight, x.astype(jnp.float32), gnn_w)

    epad = NW * KE * CH - E
    srcs = jnp.concatenate(
        [adj[0].astype(jnp.int32), jnp.arange(epad, dtype=jnp.int32) % N]
    ).reshape(NW, KE, CH)
    dsts = jnp.concatenate(
        [adj[1].astype(jnp.int32), jnp.full((epad,), N, jnp.int32)]
    ).reshape(NW, KE, CH)

    sums, degs = _segment_sum(y, srcs, dsts)

    h = _merge(sums, degs, gnn_b)

    e0s = edges[0].astype(jnp.int32).reshape(NW, KQ, CH)
    e1s = edges[1].astype(jnp.int32).reshape(NW, KQ, CH)

    hs, hd = _gatherq(h, e0s, e1s)
    return _score(hs, hd, pred_w[:, 0], pred_b)[:, 0]


# speedup vs baseline: 1.0473x; 1.0429x over previous
import jax
import jax.numpy as jnp
from jax import lax
from jax.experimental import pallas as pl
from jax.experimental.pallas import tpu as pltpu
from jax.experimental.pallas import tpu_sc as plsc

N = 10000
E = 320000
Q = 65536
D = 128
NC = 2
NS = 16
NW = NC * NS
NPAD = 10240
RT = NPAD // NS
CH = 128
EC = E // CH
KB = EC // NW
KQ = (Q // NW) // CH
QW = Q // NW
KQH = KQ // 2
QWH = QW // 2
QH = Q // 2


def _mm_body(emb_ref, x_ref, w_ref, o_ref):
    o_ref[...] = (
        jnp.dot(emb_ref[...], w_ref[0], preferred_element_type=jnp.float32)
        + jnp.dot(x_ref[...], w_ref[1], preferred_element_type=jnp.float32))


def _node_matmul(emb, x, gnn_w):
    blk = 400
    return pl.pallas_call(
        _mm_body,
        grid=(N // blk,),
        in_specs=[
            pl.BlockSpec((blk, D), lambda i: (i, 0)),
            pl.BlockSpec((blk, D), lambda i: (i, 0)),
            pl.BlockSpec((2, D, D), lambda i: (0, 0, 0)),
        ],
        out_specs=pl.BlockSpec((blk, D), lambda i: (i, 0)),
        out_shape=jax.ShapeDtypeStruct((NPAD, D), jnp.float32),
    )(emb, x, gnn_w.reshape(2, D, D))


def _seg_body(y_hbm, srcs, dsts, sum_out, deg_out,
              acc, degs, src_v, dst_v, msg0, msg1, zbuf, onesv, zv,
              sem0, sem1):
    c = lax.axis_index("c")
    s = lax.axis_index("s")
    wid = s * NC + c
    t = s

    zero16 = jnp.zeros((16,), jnp.float32)
    for i in range(8):
        for j in range(8):
            zbuf[i, pl.ds(j * 16, 16)] = zero16
    for i in range(RT // 16):
        zv[pl.ds(i * 16, 16)] = zero16
    for i in range(8):
        onesv[pl.ds(i * 16, 16)] = jnp.ones((16,), jnp.float32)
    for i in range(RT // 8):
        pltpu.sync_copy(zbuf, acc.at[pl.ds(t * RT + i * 8, 8)])
    pltpu.sync_copy(zv, degs.at[pl.ds(t * RT, RT)])
    plsc.subcore_barrier()

    base = KB * wid + jnp.maximum(wid - (NW - 4), 0)
    extra = wid >= NW - 4
    for p, (nload, nproc) in enumerate(((40, 40), (39, 38))):
        off = base + p * 40
        pltpu.sync_copy(srcs.at[pl.ds(off, nload)], src_v.at[pl.ds(0, nload)])
        pltpu.sync_copy(dsts.at[pl.ds(off, nload)], dst_v.at[pl.ds(0, nload)])

        pltpu.async_copy(y_hbm.at[src_v.at[0, 0]], msg0, sem0)
        pltpu.async_copy(y_hbm.at[src_v.at[1, 0]], msg1, sem1)
        npre = nproc if p == 0 else nproc + extra.astype(jnp.int32)

        @pl.loop(0, nproc // 2)
        def _chunks(i):
            for b, (msg, sem) in enumerate(((msg0, sem0), (msg1, sem1))):
                j = 2 * i + b
                pltpu.make_async_copy(y_hbm.at[src_v.at[j, 0]], msg,
                                      sem).wait()
                pltpu.sync_copy(msg, acc.at[dst_v.at[j, 0]], add=True)
                pltpu.sync_copy(onesv, degs.at[dst_v.at[j, 0]], add=True)

                @pl.when(j + 2 < npre)
                def _():
                    pltpu.async_copy(y_hbm.at[src_v.at[j + 2, 0]], msg, sem)

        if p == 1:
            @pl.when(extra)
            def _tail():
                pltpu.make_async_copy(y_hbm.at[src_v.at[38, 0]], msg0,
                                      sem0).wait()
                pltpu.sync_copy(msg0, acc.at[dst_v.at[38, 0]], add=True)
                pltpu.sync_copy(onesv, degs.at[dst_v.at[38, 0]], add=True)

    plsc.subcore_barrier()
    pltpu.sync_copy(acc.at[pl.ds(t * RT, RT)],
                    sum_out.at[c, pl.ds(t * RT, RT)])
    pltpu.sync_copy(degs.at[pl.ds(t * RT, RT)],
                    deg_out.at[c, pl.ds(t * RT, RT)])


def _segment_sum(y, srcs, dsts):
    mesh = plsc.VectorSubcoreMesh(core_axis_name="c", subcore_axis_name="s",
                                  num_cores=NC, num_subcores=NS)
    f = pl.kernel(
        _seg_body,
        out_type=[
            jax.ShapeDtypeStruct((NC, NPAD, D), jnp.float32),
            jax.ShapeDtypeStruct((NC, NPAD), jnp.float32),
        ],
        mesh=mesh,
        scratch_types=[
            pltpu.VMEM_SHARED((NPAD, D), jnp.float32),
            pltpu.VMEM_SHARED((NPAD,), jnp.float32),
            pltpu.VMEM((40, 1, CH), jnp.int32),
            pltpu.VMEM((40, 1, CH), jnp.int32),
            pltpu.VMEM((CH, D), jnp.float32),
            pltpu.VMEM((CH, D), jnp.float32),
            pltpu.VMEM((8, D), jnp.float32),
            pltpu.VMEM((CH,), jnp.float32),
            pltpu.VMEM((RT,), jnp.float32),
            pltpu.SemaphoreType.DMA,
            pltpu.SemaphoreType.DMA,
        ],
    )
    return f(y, srcs, dsts)


def _merge_body(s0_ref, s1_ref, d0_ref, d1_ref, b_ref, h_ref):
    deg = jnp.maximum(d0_ref[...] + d1_ref[...], 1.0)
    h_ref[...] = jnp.maximum(
        (s0_ref[...] + s1_ref[...]) / deg + b_ref[...], 0.0)


def _merge(sums, degs, gnn_b):
    blk = 512
    d2 = degs.reshape(NC, NPAD, 1)
    return pl.pallas_call(
        _merge_body,
        grid=(NPAD // blk,),
        in_specs=[
            pl.BlockSpec((blk, D), lambda i: (i, 0)),
            pl.BlockSpec((blk, D), lambda i: (i, 0)),
            pl.BlockSpec((blk, 1), lambda i: (i, 0)),
            pl.BlockSpec((blk, 1), lambda i: (i, 0)),
            pl.BlockSpec((1, D), lambda i: (0, 0)),
        ],
        out_specs=pl.BlockSpec((blk, D), lambda i: (i, 0)),
        out_shape=jax.ShapeDtypeStruct((NPAD, D), jnp.float32),
    )(sums[0], sums[1], d2[0], d2[1], gnn_b.reshape(1, D))


def _gatherq_body(h_hbm, e0s, e1s, hs_out, hd_out,
                  e0_v, e1_v, hs0, hs1, hd0, hd1,
                  sa0, sa1, sb0, sb1):
    c = lax.axis_index("c")
    s = lax.axis_index("s")
    wid = s * NC + c

    pltpu.sync_copy(e0s.at[wid], e0_v)
    pltpu.sync_copy(e1s.at[wid], e1_v)

    pltpu.async_copy(h_hbm.at[e0_v.at[0]], hs0, sa0)
    pltpu.async_copy(h_hbm.at[e1_v.at[0]], hd0, sb0)
    pltpu.async_copy(h_hbm.at[e0_v.at[1]], hs1, sa1)
    pltpu.async_copy(h_hbm.at[e1_v.at[1]], hd1, sb1)

    @pl.loop(0, KQ // 2)
    def _chunks(i):
        for b, (hs, hd, sa, sb) in enumerate(
                ((hs0, hd0, sa0, sb0), (hs1, hd1, sa1, sb1))):
            j = 2 * i + b
            base = wid * QW + j * CH
            pltpu.make_async_copy(h_hbm.at[e0_v.at[j]], hs, sa).wait()
            pltpu.make_async_copy(h_hbm.at[e1_v.at[j]], hd, sb).wait()
            pltpu.sync_copy(hs, hs_out.at[pl.ds(base, CH)])
            pltpu.sync_copy(hd, hd_out.at[pl.ds(base, CH)])

            @pl.when(j + 2 < KQ)
            def _():
                pltpu.async_copy(h_hbm.at[e0_v.at[j + 2]], hs, sa)
                pltpu.async_copy(h_hbm.at[e1_v.at[j + 2]], hd, sb)


def _gatherq(h, e0s, e1s):
    mesh = plsc.VectorSubcoreMesh(core_axis_name="c", subcore_axis_name="s",
                                  num_cores=NC, num_subcores=NS)
    f = pl.kernel(
        _gatherq_body,
        out_type=[
            jax.ShapeDtypeStruct((Q, D), jnp.float32),
            jax.ShapeDtypeStruct((Q, D), jnp.float32),
        ],
        mesh=mesh,
        scratch_types=[
            pltpu.VMEM((KQ, CH), jnp.int32),
            pltpu.VMEM((KQ, CH), jnp.int32),
            pltpu.VMEM((CH, D), jnp.float32),
            pltpu.VMEM((CH, D), jnp.float32),
            pltpu.VMEM((CH, D), jnp.float32),
            pltpu.VMEM((CH, D), jnp.float32),
            pltpu.SemaphoreType.DMA,
            pltpu.SemaphoreType.DMA,
            pltpu.SemaphoreType.DMA,
            pltpu.SemaphoreType.DMA,
        ],
    )
    return f(h, e0s, e1s)


def _score_body(hs_ref, hd_ref, w_ref, b_ref, o_ref):
    z = jnp.sum(hs_ref[...] * hd_ref[...] * w_ref[...], axis=1,
                keepdims=True) + b_ref[...]
    o_ref[...] = 1.0 / (1.0 + jnp.exp(-z))


def _score(hs, hd, pred_w, pred_b):
    blk = 16384
    return pl.pallas_call(
        _score_body,
        grid=(Q // blk,),
        in_specs=[
            pl.BlockSpec((blk, D), lambda i: (i, 0)),
            pl.BlockSpec((blk, D), lambda i: (i, 0)),
            pl.BlockSpec((1, D), lambda i: (0, 0)),
            pl.BlockSpec((1, 1), lambda i: (0, 0)),
        ],
        out_specs=pl.BlockSpec((blk, 1), lambda i: (i, 0)),
        out_shape=jax.ShapeDtypeStruct((Q, 1), jnp.float32),
    )(hs, hd, pred_w.reshape(1, D), pred_b.reshape(1, 1))


@jax.jit
def kernel(x, edges, adj, emb_weight, gnn_w, gnn_b, pred_w, pred_b):
    y = _node_matmul(emb_weight, x.astype(jnp.float32), gnn_w)

    srcs = adj[0].astype(jnp.int32).reshape(EC, 1, CH)
    dsts = adj[1].astype(jnp.int32).reshape(EC, 1, CH)

    sums, degs = _segment_sum(y, srcs, dsts)

    h = _merge(sums, degs, gnn_b)

    e0s = edges[0].astype(jnp.int32).reshape(NW, KQ, CH)
    e1s = edges[1].astype(jnp.int32).reshape(NW, KQ, CH)

    hs, hd = _gatherq(h, e0s, e1s)
    return _score(hs, hd, pred_w[:, 0], pred_b)[:, 0]


# speedup vs baseline: 1.0932x; 1.0438x over previous
import jax
import jax.numpy as jnp
from jax import lax
from jax.experimental import pallas as pl
from jax.experimental.pallas import tpu as pltpu
from jax.experimental.pallas import tpu_sc as plsc

N = 10000
E = 320000
Q = 65536
D = 128
NC = 2
NS = 16
NW = NC * NS
NPAD = 10240
RT = NPAD // NS
CH = 128
EC = E // CH
KB = EC // NW
KQ = (Q // NW) // CH
QW = Q // NW
KQH = KQ // 2
QWH = QW // 2
QH = Q // 2


def _mm_body(emb_ref, x_ref, w_ref, o_ref):
    o_ref[...] = (
        jnp.dot(emb_ref[...], w_ref[0], preferred_element_type=jnp.float32)
        + jnp.dot(x_ref[...], w_ref[1], preferred_element_type=jnp.float32))


def _node_matmul(emb, x, gnn_w):
    blk = 400
    return pl.pallas_call(
        _mm_body,
        grid=(N // blk,),
        in_specs=[
            pl.BlockSpec((blk, D), lambda i: (i, 0)),
            pl.BlockSpec((blk, D), lambda i: (i, 0)),
            pl.BlockSpec((2, D, D), lambda i: (0, 0, 0)),
        ],
        out_specs=pl.BlockSpec((blk, D), lambda i: (i, 0)),
        out_shape=jax.ShapeDtypeStruct((NPAD, D), jnp.float32),
    )(emb, x, gnn_w.reshape(2, D, D))


def _seg_body(y_hbm, adjr, sum_out, deg_out,
              acc, degs, src_v, dst_v, msg0, msg1, zbuf, onesv, zv,
              sem0, sem1):
    c = lax.axis_index("c")
    s = lax.axis_index("s")
    wid = s * NC + c
    t = s

    zero16 = jnp.zeros((16,), jnp.float32)
    for i in range(8):
        for j in range(8):
            zbuf[i, pl.ds(j * 16, 16)] = zero16
    for i in range(RT // 16):
        zv[pl.ds(i * 16, 16)] = zero16
    for i in range(8):
        onesv[pl.ds(i * 16, 16)] = jnp.ones((16,), jnp.float32)
    for i in range(RT // 8):
        pltpu.sync_copy(zbuf, acc.at[pl.ds(t * RT + i * 8, 8)])
    pltpu.sync_copy(zv, degs.at[pl.ds(t * RT, RT)])
    plsc.subcore_barrier()

    base = KB * wid + jnp.maximum(wid - (NW - 4), 0)
    extra = wid >= NW - 4
    for p, (nload, nproc) in enumerate(((40, 40), (39, 38))):
        off = base + p * 40
        pltpu.sync_copy(adjr.at[pl.ds(off, nload)], src_v.at[pl.ds(0, nload)])
        pltpu.sync_copy(adjr.at[pl.ds(EC + off, nload)],
                        dst_v.at[pl.ds(0, nload)])

        pltpu.async_copy(y_hbm.at[src_v.at[0, 0]], msg0, sem0)
        pltpu.async_copy(y_hbm.at[src_v.at[1, 0]], msg1, sem1)
        npre = nproc if p == 0 else nproc + extra.astype(jnp.int32)

        @pl.loop(0, nproc // 2)
        def _chunks(i):
            for b, (msg, sem) in enumerate(((msg0, sem0), (msg1, sem1))):
                j = 2 * i + b
                pltpu.make_async_copy(y_hbm.at[src_v.at[j, 0]], msg,
                                      sem).wait()
                pltpu.sync_copy(msg, acc.at[dst_v.at[j, 0]], add=True)
                pltpu.sync_copy(onesv, degs.at[dst_v.at[j, 0]], add=True)

                @pl.when(j + 2 < npre)
                def _():
                    pltpu.async_copy(y_hbm.at[src_v.at[j + 2, 0]], msg, sem)

        if p == 1:
            @pl.when(extra)
            def _tail():
                pltpu.make_async_copy(y_hbm.at[src_v.at[38, 0]], msg0,
                                      sem0).wait()
                pltpu.sync_copy(msg0, acc.at[dst_v.at[38, 0]], add=True)
                pltpu.sync_copy(onesv, degs.at[dst_v.at[38, 0]], add=True)

    plsc.subcore_barrier()
    pltpu.sync_copy(acc.at[pl.ds(t * RT, RT)],
                    sum_out.at[c, pl.ds(t * RT, RT)])
    pltpu.sync_copy(degs.at[pl.ds(t * RT, RT)],
                    deg_out.at[c, pl.ds(t * RT, RT)])


def _segment_sum(y, adjr):
    mesh = plsc.VectorSubcoreMesh(core_axis_name="c", subcore_axis_name="s",
                                  num_cores=NC, num_subcores=NS)
    f = pl.kernel(
        _seg_body,
        out_type=[
            jax.ShapeDtypeStruct((NC, NPAD, D), jnp.float32),
            jax.ShapeDtypeStruct((NC, NPAD), jnp.float32),
        ],
        mesh=mesh,
        scratch_types=[
            pltpu.VMEM_SHARED((NPAD, D), jnp.float32),
            pltpu.VMEM_SHARED((NPAD,), jnp.float32),
            pltpu.VMEM((40, 1, CH), jnp.int32),
            pltpu.VMEM((40, 1, CH), jnp.int32),
            pltpu.VMEM((CH, D), jnp.float32),
            pltpu.VMEM((CH, D), jnp.float32),
            pltpu.VMEM((8, D), jnp.float32),
            pltpu.VMEM((CH,), jnp.float32),
            pltpu.VMEM((RT,), jnp.float32),
            pltpu.SemaphoreType.DMA,
            pltpu.SemaphoreType.DMA,
        ],
    )
    return f(y, adjr)


def _merge_body(s_ref, d0_ref, d1_ref, b_ref, h_ref):
    deg = jnp.maximum(d0_ref[...] + d1_ref[...], 1.0)
    s = (s_ref[0] + s_ref[1]).reshape(deg.shape[0], D)
    h_ref[...] = jnp.maximum(s / deg + b_ref[...], 0.0)


def _merge(sums, degs, gnn_b):
    blk = 512
    d2 = degs.reshape(NC, NPAD, 1)
    return pl.pallas_call(
        _merge_body,
        grid=(NPAD // blk,),
        in_specs=[
            pl.BlockSpec((NC, blk, D), lambda i: (0, i, 0)),
            pl.BlockSpec((blk, 1), lambda i: (i, 0)),
            pl.BlockSpec((blk, 1), lambda i: (i, 0)),
            pl.BlockSpec((1, D), lambda i: (0, 0)),
        ],
        out_specs=pl.BlockSpec((blk, D), lambda i: (i, 0)),
        out_shape=jax.ShapeDtypeStruct((NPAD, D), jnp.float32),
    )(sums, d2[0], d2[1], gnn_b.reshape(1, D))


def _gatherq_body(h_hbm, edgr, hs_out, hd_out,
                  e0_v, e1_v, hs0, hs1, hd0, hd1,
                  sa0, sa1, sb0, sb1):
    c = lax.axis_index("c")
    s = lax.axis_index("s")
    wid = s * NC + c

    pltpu.sync_copy(edgr.at[pl.ds(wid * KQ, KQ)], e0_v)
    pltpu.sync_copy(edgr.at[pl.ds((NW + wid) * KQ, KQ)], e1_v)

    pltpu.async_copy(h_hbm.at[e0_v.at[0]], hs0, sa0)
    pltpu.async_copy(h_hbm.at[e1_v.at[0]], hd0, sb0)
    pltpu.async_copy(h_hbm.at[e0_v.at[1]], hs1, sa1)
    pltpu.async_copy(h_hbm.at[e1_v.at[1]], hd1, sb1)

    @pl.loop(0, KQ // 2)
    def _chunks(i):
        for b, (hs, hd, sa, sb) in enumerate(
                ((hs0, hd0, sa0, sb0), (hs1, hd1, sa1, sb1))):
            j = 2 * i + b
            base = wid * QW + j * CH
            pltpu.make_async_copy(h_hbm.at[e0_v.at[j]], hs, sa).wait()
            pltpu.make_async_copy(h_hbm.at[e1_v.at[j]], hd, sb).wait()
            pltpu.sync_copy(hs, hs_out.at[pl.ds(base, CH)])
            pltpu.sync_copy(hd, hd_out.at[pl.ds(base, CH)])

            @pl.when(j + 2 < KQ)
            def _():
                pltpu.async_copy(h_hbm.at[e0_v.at[j + 2]], hs, sa)
                pltpu.async_copy(h_hbm.at[e1_v.at[j + 2]], hd, sb)


def _gatherq(h, edgr):
    mesh = plsc.VectorSubcoreMesh(core_axis_name="c", subcore_axis_name="s",
                                  num_cores=NC, num_subcores=NS)
    f = pl.kernel(
        _gatherq_body,
        out_type=[
            jax.ShapeDtypeStruct((Q, D), jnp.float32),
            jax.ShapeDtypeStruct((Q, D), jnp.float32),
        ],
        mesh=mesh,
        scratch_types=[
            pltpu.VMEM((KQ, CH), jnp.int32),
            pltpu.VMEM((KQ, CH), jnp.int32),
            pltpu.VMEM((CH, D), jnp.float32),
            pltpu.VMEM((CH, D), jnp.float32),
            pltpu.VMEM((CH, D), jnp.float32),
            pltpu.VMEM((CH, D), jnp.float32),
            pltpu.SemaphoreType.DMA,
            pltpu.SemaphoreType.DMA,
            pltpu.SemaphoreType.DMA,
            pltpu.SemaphoreType.DMA,
        ],
    )
    return f(h, edgr)


def _score_body(hs_ref, hd_ref, w_ref, b_ref, o_ref):
    z = jnp.sum(hs_ref[...] * hd_ref[...] * w_ref[...], axis=1) + b_ref[0, 0]
    o_ref[...] = 1.0 / (1.0 + jnp.exp(-z))


def _score(hs, hd, pred_w, pred_b):
    blk = 16384
    return pl.pallas_call(
        _score_body,
        grid=(Q // blk,),
        in_specs=[
            pl.BlockSpec((blk, D), lambda i: (i, 0)),
            pl.BlockSpec((blk, D), lambda i: (i, 0)),
            pl.BlockSpec((1, D), lambda i: (0, 0)),
            pl.BlockSpec((1, 1), lambda i: (0, 0)),
        ],
        out_specs=pl.BlockSpec((blk,), lambda i: (i,)),
        out_shape=jax.ShapeDtypeStruct((Q,), jnp.float32),
    )(hs, hd, pred_w.reshape(1, D), pred_b.reshape(1, 1))


@jax.jit
def kernel(x, edges, adj, emb_weight, gnn_w, gnn_b, pred_w, pred_b):
    y = _node_matmul(emb_weight, x.astype(jnp.float32), gnn_w)

    adjr = adj.astype(jnp.int32).reshape(2 * EC, 1, CH)

    sums, degs = _segment_sum(y, adjr)

    h = _merge(sums, degs, gnn_b)

    edgr = edges.astype(jnp.int32).reshape(2 * NW * KQ, CH)

    hs, hd = _gatherq(h, edgr)
    return _score(hs, hd, pred_w[:, 0], pred_b)
